# baseline (speedup 1.0000x reference)
import functools

import jax
import jax.numpy as jnp
from jax import lax
from jax.experimental import pallas as pl
from jax.experimental.pallas import tpu as pltpu
from jax.experimental.pallas import tpu_sc as plsc

N_NODES = 10000
N_EDGES = 320000
D = 128

NC = 2
NS = 16
DH = D // NC

CHUNK = 96
NBUF = 4
CPW = 216
EPW = CHUNK * CPW
E_PAD = EPW * NS
RPW = 624


def _matmul_tc(x, W):
    def body(x_ref, w_ref, o_ref):
        s = jnp.dot(x_ref[...], w_ref[...], preferred_element_type=jnp.float32)
        o_ref[0] = s[:, :DH]
        o_ref[1] = s[:, DH:]

    return pl.pallas_call(
        body,
        out_shape=jax.ShapeDtypeStruct((NC, N_NODES, DH), jnp.float32),
    )(x, W)


def _concat_tc(partials):
    def body(p_ref, o_ref):
        o_ref[:, :DH] = p_ref[0]
        o_ref[:, DH:] = p_ref[1]

    grid = 10
    blk = N_NODES // grid
    return pl.pallas_call(
        body,
        grid=(grid,),
        in_specs=[pl.BlockSpec((NC, blk, DH), lambda i: (0, i, 0))],
        out_specs=pl.BlockSpec((blk, D), lambda i: (i, 0)),
        out_shape=jax.ShapeDtypeStruct((N_NODES, D), jnp.float32),
    )(partials)


def _copy_rows(src, dst, rbase, s):
    pltpu.sync_copy(src.at[pl.ds(rbase, RPW)], dst.at[pl.ds(rbase, RPW)])

    @pl.when(s == NS - 1)
    def _():
        pltpu.sync_copy(src.at[pl.ds(NS * RPW, N_NODES - NS * RPW)],
                        dst.at[pl.ds(NS * RPW, N_NODES - NS * RPW)])


def _spmv_sc(support, row, col, w):
    mesh = plsc.VectorSubcoreMesh(core_axis_name="c", subcore_axis_name="s")

    @functools.partial(
        pl.kernel,
        mesh=mesh,
        compiler_params=pltpu.CompilerParams(use_tc_tiling_on_sc=False),
        out_type=jax.ShapeDtypeStruct((N_NODES, D), jnp.float32),
        scratch_types=[
            pltpu.VMEM((NBUF, CHUNK), jnp.int32),
            pltpu.VMEM((NBUF, CHUNK), jnp.int32),
            pltpu.VMEM((NBUF, CHUNK), jnp.float32),
            pltpu.VMEM((CHUNK, DH), jnp.float32),
            pltpu.VMEM((CHUNK, DH), jnp.float32),
            pltpu.VMEM((CHUNK, DH), jnp.float32),
            pltpu.VMEM((CHUNK, DH), jnp.float32),
            pltpu.VMEM_SHARED((N_NODES, DH), jnp.float32),
            pltpu.VMEM_SHARED((N_NODES, DH), jnp.float32),
            pltpu.SemaphoreType.DMA,
            pltpu.SemaphoreType.DMA,
            pltpu.SemaphoreType.DMA,
            pltpu.SemaphoreType.DMA,
            pltpu.SemaphoreType.DMA,
            pltpu.SemaphoreType.DMA,
            pltpu.SemaphoreType.DMA,
            pltpu.SemaphoreType.DMA,
            pltpu.SemaphoreType.DMA,
            pltpu.SemaphoreType.DMA,
            pltpu.SemaphoreType.DMA,
            pltpu.SemaphoreType.DMA,
        ],
    )
    def k(support_hbm, row_hbm, col_hbm, w_hbm, out_hbm,
          colv3, rowv3, wv3, rows_0, rows_1, rows_2, rows_3,
          supp, acc, isem_0, isem_1, isem_2, isem_3,
          gsem_0, gsem_1, gsem_2, gsem_3, ssem_0, ssem_1, ssem_2, ssem_3):
        c = lax.axis_index("c")
        s = lax.axis_index("s")
        ebase = s * EPW
        rbase = s * RPW

        _copy_rows(support_hbm.at[c], supp, rbase, s)

        def zrow(i, _):
            zero = jnp.zeros((16,), jnp.float32)
            for j in range(DH // 16):
                rows_0[i, pl.ds(j * 16, 16)] = zero
            return 0
        lax.fori_loop(0, CHUNK, zrow, 0)
        nfull = RPW // CHUNK
        rem = RPW - nfull * CHUNK
        for q in range(nfull):
            pltpu.sync_copy(rows_0.at[...],
                            acc.at[pl.ds(rbase + q * CHUNK, CHUNK)])
        if rem:
            pltpu.sync_copy(rows_0.at[pl.ds(0, rem)],
                            acc.at[pl.ds(rbase + nfull * CHUNK, rem)])

        @pl.when(s == NS - 1)
        def _():
            pltpu.sync_copy(rows_0.at[pl.ds(0, N_NODES - NS * RPW)],
                            acc.at[pl.ds(NS * RPW, N_NODES - NS * RPW)])
        plsc.subcore_barrier()

        rows3 = (rows_0, rows_1, rows_2, rows_3)
        isem3 = (isem_0, isem_1, isem_2, isem_3)
        gsem3 = (gsem_0, gsem_1, gsem_2, gsem_3)
        ssem3 = (ssem_0, ssem_1, ssem_2, ssem_3)

        def issue_idx(kk, bi):
            b = ebase + kk * CHUNK
            pltpu.async_copy(col_hbm.at[pl.ds(b, CHUNK)], colv3.at[bi],
                             isem3[bi])
            pltpu.async_copy(row_hbm.at[pl.ds(b, CHUNK)], rowv3.at[bi],
                             isem3[bi])
            pltpu.async_copy(w_hbm.at[pl.ds(b, CHUNK)], wv3.at[bi],
                             isem3[bi])

        def wait_idx(kk, bi):
            b = ebase + kk * CHUNK
            pltpu.make_async_copy(col_hbm.at[pl.ds(b, CHUNK)], colv3.at[bi],
                                  isem3[bi]).wait()
            pltpu.make_async_copy(row_hbm.at[pl.ds(b, CHUNK)], rowv3.at[bi],
                                  isem3[bi]).wait()
            pltpu.make_async_copy(w_hbm.at[pl.ds(b, CHUNK)], wv3.at[bi],
                                  isem3[bi]).wait()

        def issue_gather(bi):
            pltpu.async_copy(supp.at[colv3.at[bi]], rows3[bi], gsem3[bi])

        def wait_gather(bi):
            pltpu.make_async_copy(supp.at[colv3.at[bi]], rows3[bi],
                                  gsem3[bi]).wait()

        def issue_scatter(bi):
            pltpu.async_copy(rows3[bi], acc.at[rowv3.at[bi]], ssem3[bi],
                             add=True)

        def wait_scatter(bi):
            pltpu.make_async_copy(rows3[bi], acc.at[rowv3.at[bi]],
                                  ssem3[bi]).wait()

        issue_idx(0, 0)
        issue_idx(1, 1)
        wait_idx(0, 0)
        issue_gather(0)

        def step(kk, cur):
            nxt = (cur + 1) % NBUF
            nx2 = (cur + 2) % NBUF

            @pl.when(kk >= 2)
            def _():
                wait_scatter(nx2)

            @pl.when(kk + 2 < CPW)
            def _():
                issue_idx(kk + 2, nx2)

            @pl.when(kk + 1 < CPW)
            def _():
                wait_idx(kk + 1, nxt)
                issue_gather(nxt)

            wait_gather(cur)

            for g in range(CHUNK // 16):
                wvec = wv3[cur, pl.ds(g * 16, 16)]
                for i in range(16):
                    e = g * 16 + i
                    wb = wvec[i]
                    for j in range(DH // 16):
                        v = rows3[cur][e, pl.ds(j * 16, 16)]
                        rows3[cur][e, pl.ds(j * 16, 16)] = v * wb

            issue_scatter(cur)

        def chunk_quad(t, _):
            step(NBUF * t, 0)
            step(NBUF * t + 1, 1)
            step(NBUF * t + 2, 2)
            step(NBUF * t + 3, 3)
            return 0
        lax.fori_loop(0, CPW // NBUF, chunk_quad, 0)
        wait_scatter((CPW - 2) % NBUF)
        wait_scatter((CPW - 1) % NBUF)

        plsc.subcore_barrier()
        pltpu.sync_copy(acc.at[pl.ds(rbase, RPW)],
                        out_hbm.at[pl.ds(rbase, RPW), pl.ds(c * DH, DH)])

        @pl.when(s == NS - 1)
        def _():
            pltpu.sync_copy(
                acc.at[pl.ds(NS * RPW, N_NODES - NS * RPW)],
                out_hbm.at[pl.ds(NS * RPW, N_NODES - NS * RPW),
                           pl.ds(c * DH, DH)])

    return k(support, row, col, w)


def kernel(x, edge_index, edge_weight, W):
    support = _matmul_tc(x, W)

    row = edge_index[0].astype(jnp.int32)
    col = edge_index[1].astype(jnp.int32)
    w = edge_weight.astype(jnp.float32)

    pad = E_PAD - N_EDGES
    zi = jnp.zeros((pad,), jnp.int32)
    row = jnp.concatenate([row, zi])
    col = jnp.concatenate([col, zi])
    w = jnp.concatenate([w, jnp.zeros((pad,), jnp.float32)])

    return _spmv_sc(support, row, col, w)


# speedup vs baseline: 1.0567x; 1.0567x over previous
import functools

import jax
import jax.numpy as jnp
from jax import lax
from jax.experimental import pallas as pl
from jax.experimental.pallas import tpu as pltpu
from jax.experimental.pallas import tpu_sc as plsc

N_NODES = 10000
N_EDGES = 320000
D = 128

NC = 2
NS = 16
DH = D // NC

CHUNK = 64
NBUF = 4
CPW = 324
EPW = CHUNK * CPW
E_PAD = EPW * NS
RPW = 624


def _matmul_tc(x, W):
    def body(x_ref, w_ref, o_ref):
        s = jnp.dot(x_ref[...], w_ref[...], preferred_element_type=jnp.float32)
        o_ref[0] = s[:, :DH]
        o_ref[1] = s[:, DH:]

    return pl.pallas_call(
        body,
        out_shape=jax.ShapeDtypeStruct((NC, N_NODES, DH), jnp.float32),
    )(x, W)


def _copy_rows(src, dst, rbase, s):
    pltpu.sync_copy(src.at[pl.ds(rbase, RPW)], dst.at[pl.ds(rbase, RPW)])

    @pl.when(s == NS - 1)
    def _():
        pltpu.sync_copy(src.at[pl.ds(NS * RPW, N_NODES - NS * RPW)],
                        dst.at[pl.ds(NS * RPW, N_NODES - NS * RPW)])


def _spmv_sc(support, row, col, w):
    mesh = plsc.VectorSubcoreMesh(core_axis_name="c", subcore_axis_name="s")

    @functools.partial(
        pl.kernel,
        mesh=mesh,
        compiler_params=pltpu.CompilerParams(use_tc_tiling_on_sc=False),
        out_type=jax.ShapeDtypeStruct((N_NODES, D), jnp.float32),
        scratch_types=[
            pltpu.VMEM((NBUF, CHUNK), jnp.int32),
            pltpu.VMEM((NBUF, CHUNK), jnp.int32),
            pltpu.VMEM((NBUF, CHUNK), jnp.float32),
            pltpu.VMEM((CHUNK, DH), jnp.float32),
            pltpu.VMEM((CHUNK, DH), jnp.float32),
            pltpu.VMEM((CHUNK, DH), jnp.float32),
            pltpu.VMEM((CHUNK, DH), jnp.float32),
            pltpu.VMEM_SHARED((N_NODES, DH), jnp.float32),
            pltpu.VMEM_SHARED((N_NODES, DH), jnp.float32),
            pltpu.SemaphoreType.DMA,
            pltpu.SemaphoreType.DMA,
            pltpu.SemaphoreType.DMA,
            pltpu.SemaphoreType.DMA,
            pltpu.SemaphoreType.DMA,
            pltpu.SemaphoreType.DMA,
            pltpu.SemaphoreType.DMA,
            pltpu.SemaphoreType.DMA,
            pltpu.SemaphoreType.DMA,
            pltpu.SemaphoreType.DMA,
            pltpu.SemaphoreType.DMA,
            pltpu.SemaphoreType.DMA,
        ],
    )
    def k(support_hbm, row_hbm, col_hbm, w_hbm, out_hbm,
          colv3, rowv3, wv3, rows_0, rows_1, rows_2, rows_3,
          supp, acc, isem_0, isem_1, isem_2, isem_3,
          gsem_0, gsem_1, gsem_2, gsem_3, ssem_0, ssem_1, ssem_2, ssem_3):
        c = lax.axis_index("c")
        s = lax.axis_index("s")
        ebase = s * EPW
        rbase = s * RPW

        _copy_rows(support_hbm.at[c], supp, rbase, s)

        def zrow(i, _):
            zero = jnp.zeros((16,), jnp.float32)
            for j in range(DH // 16):
                rows_0[i, pl.ds(j * 16, 16)] = zero
            return 0
        lax.fori_loop(0, CHUNK, zrow, 0)
        nfull = RPW // CHUNK
        rem = RPW - nfull * CHUNK
        for q in range(nfull):
            pltpu.sync_copy(rows_0.at[...],
                            acc.at[pl.ds(rbase + q * CHUNK, CHUNK)])
        if rem:
            pltpu.sync_copy(rows_0.at[pl.ds(0, rem)],
                            acc.at[pl.ds(rbase + nfull * CHUNK, rem)])

        @pl.when(s == NS - 1)
        def _():
            pltpu.sync_copy(rows_0.at[pl.ds(0, N_NODES - NS * RPW)],
                            acc.at[pl.ds(NS * RPW, N_NODES - NS * RPW)])
        plsc.subcore_barrier()

        rows3 = (rows_0, rows_1, rows_2, rows_3)
        isem3 = (isem_0, isem_1, isem_2, isem_3)
        gsem3 = (gsem_0, gsem_1, gsem_2, gsem_3)
        ssem3 = (ssem_0, ssem_1, ssem_2, ssem_3)

        def issue_idx(kk, bi):
            b = ebase + kk * CHUNK
            pltpu.async_copy(col_hbm.at[pl.ds(b, CHUNK)], colv3.at[bi],
                             isem3[bi])
            pltpu.async_copy(row_hbm.at[pl.ds(b, CHUNK)], rowv3.at[bi],
                             isem3[bi])
            pltpu.async_copy(w_hbm.at[pl.ds(b, CHUNK)], wv3.at[bi],
                             isem3[bi])

        def wait_idx(kk, bi):
            b = ebase + kk * CHUNK
            pltpu.make_async_copy(col_hbm.at[pl.ds(b, CHUNK)], colv3.at[bi],
                                  isem3[bi]).wait()
            pltpu.make_async_copy(row_hbm.at[pl.ds(b, CHUNK)], rowv3.at[bi],
                                  isem3[bi]).wait()
            pltpu.make_async_copy(w_hbm.at[pl.ds(b, CHUNK)], wv3.at[bi],
                                  isem3[bi]).wait()

        def issue_gather(bi):
            pltpu.async_copy(supp.at[colv3.at[bi]], rows3[bi], gsem3[bi])

        def wait_gather(bi):
            pltpu.make_async_copy(supp.at[colv3.at[bi]], rows3[bi],
                                  gsem3[bi]).wait()

        def issue_scatter(bi):
            pltpu.async_copy(rows3[bi], acc.at[rowv3.at[bi]], ssem3[bi],
                             add=True)

        def wait_scatter(bi):
            pltpu.make_async_copy(rows3[bi], acc.at[rowv3.at[bi]],
                                  ssem3[bi]).wait()

        issue_idx(0, 0)
        issue_idx(1, 1)
        wait_idx(0, 0)
        issue_gather(0)

        def step(kk, cur):
            nxt = (cur + 1) % NBUF
            nx2 = (cur + 2) % NBUF

            @pl.when(kk >= 2)
            def _():
                wait_scatter(nx2)

            @pl.when(kk + 2 < CPW)
            def _():
                issue_idx(kk + 2, nx2)

            @pl.when(kk + 1 < CPW)
            def _():
                wait_idx(kk + 1, nxt)
                issue_gather(nxt)

            wait_gather(cur)

            for g in range(CHUNK // 16):
                wvec = wv3[cur, pl.ds(g * 16, 16)]
                for i in range(16):
                    e = g * 16 + i
                    wb = wvec[i]
                    for j in range(DH // 16):
                        v = rows3[cur][e, pl.ds(j * 16, 16)]
                        rows3[cur][e, pl.ds(j * 16, 16)] = v * wb

            issue_scatter(cur)

        def chunk_quad(t, _):
            step(NBUF * t, 0)
            step(NBUF * t + 1, 1)
            step(NBUF * t + 2, 2)
            step(NBUF * t + 3, 3)
            return 0
        lax.fori_loop(0, CPW // NBUF, chunk_quad, 0)
        wait_scatter((CPW - 2) % NBUF)
        wait_scatter((CPW - 1) % NBUF)

        plsc.subcore_barrier()
        pltpu.sync_copy(acc.at[pl.ds(rbase, RPW)],
                        out_hbm.at[pl.ds(rbase, RPW), pl.ds(c * DH, DH)])

        @pl.when(s == NS - 1)
        def _():
            pltpu.sync_copy(
                acc.at[pl.ds(NS * RPW, N_NODES - NS * RPW)],
                out_hbm.at[pl.ds(NS * RPW, N_NODES - NS * RPW),
                           pl.ds(c * DH, DH)])

    return k(support, row, col, w)


def kernel(x, edge_index, edge_weight, W):
    support = _matmul_tc(x, W)

    row = edge_index[0].astype(jnp.int32)
    col = edge_index[1].astype(jnp.int32)
    w = edge_weight.astype(jnp.float32)

    pad = E_PAD - N_EDGES
    zi = jnp.zeros((pad,), jnp.int32)
    row = jnp.concatenate([row, zi])
    col = jnp.concatenate([col, zi])
    w = jnp.concatenate([w, jnp.zeros((pad,), jnp.float32)])

    return _spmv_sc(support, row, col, w)
